# baseline (speedup 1.0000x reference)
import functools

import jax
import jax.numpy as jnp
from jax import lax
from jax.experimental import pallas as pl
from jax.experimental.pallas import tpu as pltpu
from jax.experimental.pallas import tpu_sc as plsc

_N = 10000
_E = 320000
_IN = 128
_OUT = 128
_R = 16
_B = 4

_TN = 400
_NT = _N // _TN

_NC = 2
_NS = 16
_NW = _NC * _NS
_EPW = 10240
_EPAD = _NW * _EPW - _E
_W = 64
_KW = _EPW // _W
_SW = 20
_NSC = _KW // _SW
_ND = 3
_NPAD = 10240
_RPS = _NPAD // _NS


def _h_body(x_ref, bases_ref, coeff_ref, wselft_ref, b_ref, h_ref, self_ref,
            wbig_ref):
    n = pl.program_id(0)

    @pl.when(n == 0)
    def _():
        for r in range(_R):
            w = (coeff_ref[r, 0] * bases_ref[0]
                 + coeff_ref[r, 1] * bases_ref[1]
                 + coeff_ref[r, 2] * bases_ref[2]
                 + coeff_ref[r, 3] * bases_ref[3])
            wbig_ref[:, r * _OUT:(r + 1) * _OUT] = w.astype(jnp.bfloat16)
        wbig_ref[:, _R * _OUT:] = wselft_ref[...].astype(jnp.bfloat16)

    x = x_ref[...].astype(jnp.bfloat16)
    h = lax.dot_general(x, wbig_ref[...], (((1,), (0,)), ((), ())),
                        preferred_element_type=jnp.float32)
    for r in range(_R):
        h_ref[r] = h[:, r * _OUT:(r + 1) * _OUT]
    self_ref[...] = h[:, _R * _OUT:] + b_ref[...]


def _h_pallas(x, bases, coeff, wself_t, b2d):
    return pl.pallas_call(
        _h_body,
        grid=(_NT,),
        in_specs=[
            pl.BlockSpec((_TN, _IN), lambda n: (n, 0)),
            pl.BlockSpec((_B, _IN, _OUT), lambda n: (0, 0, 0)),
            pl.BlockSpec(memory_space=pltpu.SMEM),
            pl.BlockSpec((_IN, _OUT), lambda n: (0, 0)),
            pl.BlockSpec((1, _OUT), lambda n: (0, 0)),
        ],
        out_specs=[
            pl.BlockSpec((_R, _TN, _OUT), lambda n: (0, n, 0)),
            pl.BlockSpec((_TN, _OUT), lambda n: (n, 0)),
        ],
        out_shape=[
            jax.ShapeDtypeStruct((_R, _N, _OUT), jnp.float32),
            jax.ShapeDtypeStruct((_N, _OUT), jnp.float32),
        ],
        scratch_shapes=[
            pltpu.VMEM((_IN, (_R + 1) * _OUT), jnp.bfloat16),
        ],
    )(x, bases, coeff, wself_t, b2d)


def _sc_body(h_hbm, ei_hbm, et_hbm, out_hbm,
             idx_v, et_v, tgt_v, gb0, gb1, gb2, acc_sh, sm0, sm1, sm2):
    c = lax.axis_index("c")
    s = lax.axis_index("s")
    wid = s * _NC + c
    bufs = [gb0, gb1, gb2]
    sems = [sm0, sm1, sm2]

    @pl.loop(0, _W)
    def _(rr):
        for cc in range(_OUT // 16):
            gb0[rr, pl.ds(cc * 16, 16)] = jnp.zeros((16,), jnp.float32)

    @pl.loop(0, _RPS // _W)
    def _(k):
        pltpu.sync_copy(gb0, acc_sh.at[pl.ds(s * _RPS + k * _W, _W)])
    plsc.subcore_barrier()

    @pl.loop(0, _NSC)
    def _(q):
        pltpu.sync_copy(ei_hbm.at[0, wid, q], idx_v)
        pltpu.sync_copy(et_hbm.at[wid, q], et_v)
        pltpu.sync_copy(ei_hbm.at[1, wid, q], tgt_v)

        @pl.loop(0, _SW)
        def _(j):
            for cc in range(_W // 16):
                sl = pl.ds(cc * 16, 16)
                idx_v[j, sl] = idx_v[j, sl] + et_v[j, sl] * _N

        for b in range(_ND):
            pltpu.async_copy(h_hbm.at[idx_v.at[b]], bufs[b], sems[b])

        @pl.loop(0, _SW // _ND)
        def _(g):
            for b in range(_ND):
                j = _ND * g + b
                pltpu.make_async_copy(h_hbm.at[idx_v.at[j]], bufs[b],
                                      sems[b]).wait()
                pltpu.sync_copy(bufs[b], acc_sh.at[tgt_v.at[j]], add=True)

                @pl.when(j + _ND < _SW)
                def _(b=b, j=j):
                    pltpu.async_copy(h_hbm.at[idx_v.at[j + _ND]], bufs[b],
                                     sems[b])

        for j in range(_ND * (_SW // _ND), _SW):
            b = j % _ND
            pltpu.make_async_copy(h_hbm.at[idx_v.at[j]], bufs[b],
                                  sems[b]).wait()
            pltpu.sync_copy(bufs[b], acc_sh.at[tgt_v.at[j]], add=True)

    plsc.subcore_barrier()

    pltpu.sync_copy(acc_sh.at[pl.ds(s * _RPS, _RPS)],
                    out_hbm.at[c].at[pl.ds(s * _RPS, _RPS)])


_sc_scatter = functools.partial(
    pl.kernel,
    out_type=jax.ShapeDtypeStruct((_NC, _NPAD, _OUT), jnp.float32),
    mesh=plsc.VectorSubcoreMesh(core_axis_name="c", subcore_axis_name="s"),
    scratch_types=[
        pltpu.VMEM((_SW, _W), jnp.int32),
        pltpu.VMEM((_SW, _W), jnp.int32),
        pltpu.VMEM((_SW, _W), jnp.int32),
        pltpu.VMEM((_W, _OUT), jnp.float32),
        pltpu.VMEM((_W, _OUT), jnp.float32),
        pltpu.VMEM((_W, _OUT), jnp.float32),
        pltpu.VMEM_SHARED((_NPAD, _OUT), jnp.float32),
        pltpu.SemaphoreType.DMA,
        pltpu.SemaphoreType.DMA,
        pltpu.SemaphoreType.DMA,
    ],
)(_sc_body)


def _relu_body(self_ref, acc_ref, out_ref):
    out_ref[...] = jnp.maximum(self_ref[...] + acc_ref[0] + acc_ref[1], 0.0)


_TR = 2000


def _relu_pallas(self_out, acc):
    return pl.pallas_call(
        _relu_body,
        grid=(_N // _TR,),
        in_specs=[
            pl.BlockSpec((_TR, _OUT), lambda n: (n, 0)),
            pl.BlockSpec((_NC, _TR, _OUT), lambda n: (0, n, 0)),
        ],
        out_specs=pl.BlockSpec((_TR, _OUT), lambda n: (n, 0)),
        out_shape=jax.ShapeDtypeStruct((_N, _OUT), jnp.float32),
    )(self_out, acc)


def kernel(node_features, edge_index, edge_type, W_self_w, W_self_b,
           bases, coefficients):
    h, self_out = _h_pallas(node_features, bases, coefficients,
                            W_self_w.T, W_self_b.reshape(1, _OUT))
    pad_src = jnp.zeros((1, _EPAD), jnp.int32)
    pad_tgt = _N + (jnp.arange(_EPAD, dtype=jnp.int32) % (_NPAD - _N))[None, :]
    ei_pad = jnp.concatenate([pad_src, pad_tgt], axis=0)
    ei = jnp.concatenate([edge_index, ei_pad], axis=1).reshape(
        2, _NW, _NSC, _SW, _W)
    et = jnp.concatenate(
        [edge_type, jnp.zeros((_EPAD,), jnp.int32)]).reshape(
        _NW, _NSC, _SW, _W)
    acc = _sc_scatter(h.reshape(_R * _N, _OUT), ei, et)
    return _relu_pallas(self_out, acc)


# speedup vs baseline: 2.2643x; 2.2643x over previous
import functools

import jax
import jax.numpy as jnp
from jax import lax
from jax.experimental import pallas as pl
from jax.experimental.pallas import tpu as pltpu
from jax.experimental.pallas import tpu_sc as plsc

_N = 10000
_E = 320000
_IN = 128
_OUT = 128
_R = 16
_B = 4

_TN = 400
_NT = _N // _TN

_NC = 2
_NS = 16
_NW = _NC * _NS
_EPW = 10240
_EPAD = _NW * _EPW - _E
_W = 64
_KW = _EPW // _W
_SW = 20
_NSC = _KW // _SW
_ND = 3
_NPAD = 10240
_RPS = _NPAD // _NS


def _h_body(x_ref, bases_ref, coeff_ref, wselft_ref, b_ref, h_ref, self_ref,
            wbig_ref):
    n = pl.program_id(0)

    @pl.when(n == 0)
    def _():
        for r in range(_R):
            w = (coeff_ref[r, 0] * bases_ref[0]
                 + coeff_ref[r, 1] * bases_ref[1]
                 + coeff_ref[r, 2] * bases_ref[2]
                 + coeff_ref[r, 3] * bases_ref[3])
            wbig_ref[:, r * _OUT:(r + 1) * _OUT] = w.astype(jnp.bfloat16)
        wbig_ref[:, _R * _OUT:] = wselft_ref[...].astype(jnp.bfloat16)

    x = x_ref[...].astype(jnp.bfloat16)
    h = lax.dot_general(x, wbig_ref[...], (((1,), (0,)), ((), ())),
                        preferred_element_type=jnp.float32)
    for r in range(_R):
        h_ref[r] = h[:, r * _OUT:(r + 1) * _OUT]
    self_ref[...] = h[:, _R * _OUT:] + b_ref[...]


def _h_pallas(x, bases, coeff, wself_t, b2d):
    return pl.pallas_call(
        _h_body,
        grid=(_NT,),
        in_specs=[
            pl.BlockSpec((_TN, _IN), lambda n: (n, 0)),
            pl.BlockSpec((_B, _IN, _OUT), lambda n: (0, 0, 0)),
            pl.BlockSpec(memory_space=pltpu.SMEM),
            pl.BlockSpec((_IN, _OUT), lambda n: (0, 0)),
            pl.BlockSpec((1, _OUT), lambda n: (0, 0)),
        ],
        out_specs=[
            pl.BlockSpec((_R, _TN, _OUT), lambda n: (0, n, 0)),
            pl.BlockSpec((_TN, _OUT), lambda n: (n, 0)),
        ],
        out_shape=[
            jax.ShapeDtypeStruct((_R, _N, _OUT), jnp.float32),
            jax.ShapeDtypeStruct((_N, _OUT), jnp.float32),
        ],
        scratch_shapes=[
            pltpu.VMEM((_IN, (_R + 1) * _OUT), jnp.bfloat16),
        ],
    )(x, bases, coeff, wself_t, b2d)


def _sc_body(h_hbm, ei_hbm, et_hbm, out_hbm,
             idx_v, et_v, tgt_v, gb0, gb1, gb2, acc_sh, sm0, sm1, sm2):
    c = lax.axis_index("c")
    s = lax.axis_index("s")
    wid = s * _NC + c
    bufs = [gb0, gb1, gb2]
    sems = [sm0, sm1, sm2]

    @pl.loop(0, _W)
    def _(rr):
        for cc in range(_OUT // 16):
            gb0[rr, pl.ds(cc * 16, 16)] = jnp.zeros((16,), jnp.float32)

    @pl.loop(0, _RPS // _W)
    def _(k):
        pltpu.sync_copy(gb0, acc_sh.at[pl.ds(s * _RPS + k * _W, _W)])
    plsc.subcore_barrier()

    @pl.loop(0, _NSC)
    def _(q):
        pltpu.sync_copy(ei_hbm.at[0, wid, q], idx_v)
        pltpu.sync_copy(et_hbm.at[wid, q], et_v)
        pltpu.sync_copy(ei_hbm.at[1, wid, q], tgt_v)

        @pl.loop(0, _SW)
        def _(j):
            for cc in range(_W // 16):
                sl = pl.ds(cc * 16, 16)
                idx_v[j, sl] = idx_v[j, sl] + et_v[j, sl] * _N

        for b in range(_ND):
            pltpu.async_copy(h_hbm.at[idx_v.at[b]], bufs[b], sems[b])

        @pl.loop(0, _SW // _ND)
        def _(g):
            for b in range(_ND):
                j = _ND * g + b
                pltpu.make_async_copy(h_hbm.at[idx_v.at[j]], bufs[b],
                                      sems[b]).wait()
                pltpu.sync_copy(bufs[b], acc_sh.at[tgt_v.at[j]], add=True)

                @pl.when(j + _ND < _SW)
                def _(b=b, j=j):
                    pltpu.async_copy(h_hbm.at[idx_v.at[j + _ND]], bufs[b],
                                     sems[b])

        for j in range(_ND * (_SW // _ND), _SW):
            b = j % _ND
            pltpu.make_async_copy(h_hbm.at[idx_v.at[j]], bufs[b],
                                  sems[b]).wait()
            pltpu.sync_copy(bufs[b], acc_sh.at[tgt_v.at[j]], add=True)

    plsc.subcore_barrier()

    pltpu.sync_copy(acc_sh.at[pl.ds(s * _RPS, _RPS)],
                    out_hbm.at[c].at[pl.ds(s * _RPS, _RPS)])


_sc_scatter = functools.partial(
    pl.kernel,
    out_type=jax.ShapeDtypeStruct((_NC, _NPAD, _OUT), jnp.float32),
    mesh=plsc.VectorSubcoreMesh(core_axis_name="c", subcore_axis_name="s"),
    scratch_types=[
        pltpu.VMEM((_SW, _W), jnp.int32),
        pltpu.VMEM((_SW, _W), jnp.int32),
        pltpu.VMEM((_SW, _W), jnp.int32),
        pltpu.VMEM((_W, _OUT), jnp.float32),
        pltpu.VMEM((_W, _OUT), jnp.float32),
        pltpu.VMEM((_W, _OUT), jnp.float32),
        pltpu.VMEM_SHARED((_NPAD, _OUT), jnp.float32),
        pltpu.SemaphoreType.DMA,
        pltpu.SemaphoreType.DMA,
        pltpu.SemaphoreType.DMA,
    ],
)(_sc_body)


def _relu_body(self_ref, acc_ref, out_ref):
    out_ref[...] = jnp.maximum(self_ref[...] + acc_ref[0] + acc_ref[1], 0.0)


_TR = 2000


def _relu_pallas(self_out, acc):
    return pl.pallas_call(
        _relu_body,
        grid=(_N // _TR,),
        in_specs=[
            pl.BlockSpec((_TR, _OUT), lambda n: (n, 0)),
            pl.BlockSpec((_NC, _TR, _OUT), lambda n: (0, n, 0)),
        ],
        out_specs=pl.BlockSpec((_TR, _OUT), lambda n: (n, 0)),
        out_shape=jax.ShapeDtypeStruct((_N, _OUT), jnp.float32),
    )(self_out, acc)


def kernel(node_features, edge_index, edge_type, W_self_w, W_self_b,
           bases, coefficients):
    h, self_out = _h_pallas(node_features, bases, coefficients,
                            W_self_w.T, W_self_b.reshape(1, _OUT))
    ppw = _EPAD // _NW
    pad1 = jnp.arange(ppw, dtype=jnp.int32)[None, :]
    pad_src = jnp.broadcast_to(pad1, (_NW, ppw))
    pad_tgt = jnp.broadcast_to(_N + pad1, (_NW, ppw))
    pad_et = jnp.zeros((_NW, ppw), jnp.int32)
    src = jnp.concatenate(
        [edge_index[0].reshape(_NW, -1), pad_src], axis=1)
    tgt = jnp.concatenate(
        [edge_index[1].reshape(_NW, -1), pad_tgt], axis=1)
    ei = jnp.stack([src, tgt]).reshape(2, _NW, _NSC, _SW, _W)
    et = jnp.concatenate(
        [edge_type.reshape(_NW, -1), pad_et], axis=1).reshape(
        _NW, _NSC, _SW, _W)
    acc = _sc_scatter(h.reshape(_R * _N, _OUT), ei, et)
    return _relu_pallas(self_out, acc)


# speedup vs baseline: 2.4976x; 1.1031x over previous
import functools

import jax
import jax.numpy as jnp
from jax import lax
from jax.experimental import pallas as pl
from jax.experimental.pallas import tpu as pltpu
from jax.experimental.pallas import tpu_sc as plsc

_N = 10000
_E = 320000
_IN = 128
_OUT = 128
_R = 16
_B = 4

_TN = 400
_NT = _N // _TN

_NC = 2
_NS = 16
_NW = _NC * _NS
_EPW = 10000
_W = 80
_HW = _W // 2
_KW = _EPW // _W
_SW = 25
_NSC = _KW // _SW
_ND = 2
_NPAD = 10240
_RPS = _NPAD // _NS


def _h_body(x_ref, bases_ref, coeff_ref, wselft_ref, b_ref, h_ref, self_ref,
            wbig_ref):
    n = pl.program_id(0)

    @pl.when(n == 0)
    def _():
        for r in range(_R):
            w = (coeff_ref[r, 0] * bases_ref[0]
                 + coeff_ref[r, 1] * bases_ref[1]
                 + coeff_ref[r, 2] * bases_ref[2]
                 + coeff_ref[r, 3] * bases_ref[3])
            wbig_ref[:, r * _OUT:(r + 1) * _OUT] = w.astype(jnp.bfloat16)
        wbig_ref[:, _R * _OUT:] = wselft_ref[...].astype(jnp.bfloat16)

    x = x_ref[...].astype(jnp.bfloat16)
    h = lax.dot_general(x, wbig_ref[...], (((1,), (0,)), ((), ())),
                        preferred_element_type=jnp.float32)
    for r in range(_R):
        h_ref[r] = h[:, r * _OUT:(r + 1) * _OUT]
    self_ref[...] = h[:, _R * _OUT:] + b_ref[...]


def _h_pallas(x, bases, coeff, wself_t, b2d):
    return pl.pallas_call(
        _h_body,
        grid=(_NT,),
        in_specs=[
            pl.BlockSpec((_TN, _IN), lambda n: (n, 0)),
            pl.BlockSpec((_B, _IN, _OUT), lambda n: (0, 0, 0)),
            pl.BlockSpec(memory_space=pltpu.SMEM),
            pl.BlockSpec((_IN, _OUT), lambda n: (0, 0)),
            pl.BlockSpec((1, _OUT), lambda n: (0, 0)),
        ],
        out_specs=[
            pl.BlockSpec((_R, _TN, _OUT), lambda n: (0, n, 0)),
            pl.BlockSpec((_TN, _OUT), lambda n: (n, 0)),
        ],
        out_shape=[
            jax.ShapeDtypeStruct((_R, _N, _OUT), jnp.float32),
            jax.ShapeDtypeStruct((_N, _OUT), jnp.float32),
        ],
        scratch_shapes=[
            pltpu.VMEM((_IN, (_R + 1) * _OUT), jnp.bfloat16),
        ],
    )(x, bases, coeff, wself_t, b2d)


def _sc_body(h_hbm, ei_hbm, et_hbm, out_hbm,
             idx_v, et_v, tgt_v, gb0, gb1, acc_sh,
             sm00, sm01, sm10, sm11):
    c = lax.axis_index("c")
    s = lax.axis_index("s")
    wid = s * _NC + c
    bufs = [gb0, gb1]
    sems = [(sm00, sm01), (sm10, sm11)]

    def _start_gather(j, b):
        pltpu.async_copy(h_hbm.at[idx_v.at[j, pl.ds(0, _HW)]],
                         bufs[b].at[pl.ds(0, _HW)], sems[b][0])
        pltpu.async_copy(h_hbm.at[idx_v.at[j, pl.ds(_HW, _HW)]],
                         bufs[b].at[pl.ds(_HW, _HW)], sems[b][1])

    def _finish_window(j, b):
        pltpu.make_async_copy(h_hbm.at[idx_v.at[j, pl.ds(0, _HW)]],
                              bufs[b].at[pl.ds(0, _HW)], sems[b][0]).wait()
        pltpu.make_async_copy(h_hbm.at[idx_v.at[j, pl.ds(_HW, _HW)]],
                              bufs[b].at[pl.ds(_HW, _HW)], sems[b][1]).wait()
        pltpu.sync_copy(bufs[b], acc_sh.at[tgt_v.at[j]], add=True)

    @pl.loop(0, _W)
    def _(rr):
        for cc in range(_OUT // 16):
            gb0[rr, pl.ds(cc * 16, 16)] = jnp.zeros((16,), jnp.float32)

    @pl.loop(0, _RPS // _W)
    def _(k):
        pltpu.sync_copy(gb0, acc_sh.at[pl.ds(s * _RPS + k * _W, _W)])
    plsc.subcore_barrier()

    @pl.loop(0, _NSC)
    def _(q):
        pltpu.sync_copy(ei_hbm.at[0, wid, q], idx_v)
        pltpu.sync_copy(et_hbm.at[wid, q], et_v)
        pltpu.sync_copy(ei_hbm.at[1, wid, q], tgt_v)

        @pl.loop(0, _SW)
        def _(j):
            for cc in range(_W // 16):
                sl = pl.ds(cc * 16, 16)
                idx_v[j, sl] = idx_v[j, sl] + et_v[j, sl] * _N

        for b in range(_ND):
            _start_gather(b, b)

        @pl.loop(0, _SW // _ND)
        def _(g):
            for b in range(_ND):
                j = _ND * g + b
                _finish_window(j, b)

                @pl.when(j + _ND < _SW)
                def _(b=b, j=j):
                    _start_gather(j + _ND, b)

        for j in range(_ND * (_SW // _ND), _SW):
            _finish_window(j, j % _ND)

    plsc.subcore_barrier()

    pltpu.sync_copy(acc_sh.at[pl.ds(s * _RPS, _RPS)],
                    out_hbm.at[c].at[pl.ds(s * _RPS, _RPS)])


_sc_scatter = functools.partial(
    pl.kernel,
    out_type=jax.ShapeDtypeStruct((_NC, _NPAD, _OUT), jnp.float32),
    mesh=plsc.VectorSubcoreMesh(core_axis_name="c", subcore_axis_name="s"),
    scratch_types=[
        pltpu.VMEM((_SW, _W), jnp.int32),
        pltpu.VMEM((_SW, _W), jnp.int32),
        pltpu.VMEM((_SW, _W), jnp.int32),
        pltpu.VMEM((_W, _OUT), jnp.float32),
        pltpu.VMEM((_W, _OUT), jnp.float32),
        pltpu.VMEM_SHARED((_NPAD, _OUT), jnp.float32),
        pltpu.SemaphoreType.DMA,
        pltpu.SemaphoreType.DMA,
        pltpu.SemaphoreType.DMA,
        pltpu.SemaphoreType.DMA,
    ],
)(_sc_body)


def _relu_body(self_ref, acc_ref, out_ref):
    out_ref[...] = jnp.maximum(self_ref[...] + acc_ref[0] + acc_ref[1], 0.0)


_TR = 2000


def _relu_pallas(self_out, acc):
    return pl.pallas_call(
        _relu_body,
        grid=(_N // _TR,),
        in_specs=[
            pl.BlockSpec((_TR, _OUT), lambda n: (n, 0)),
            pl.BlockSpec((_NC, _TR, _OUT), lambda n: (0, n, 0)),
        ],
        out_specs=pl.BlockSpec((_TR, _OUT), lambda n: (n, 0)),
        out_shape=jax.ShapeDtypeStruct((_N, _OUT), jnp.float32),
    )(self_out, acc)


def kernel(node_features, edge_index, edge_type, W_self_w, W_self_b,
           bases, coefficients):
    h, self_out = _h_pallas(node_features, bases, coefficients,
                            W_self_w.T, W_self_b.reshape(1, _OUT))
    ei = edge_index.reshape(2, _NW, _NSC, _SW, _W)
    et = edge_type.reshape(_NW, _NSC, _SW, _W)
    acc = _sc_scatter(h.reshape(_R * _N, _OUT), ei, et)
    return _relu_pallas(self_out, acc)


# speedup vs baseline: 2.7244x; 1.0908x over previous
import functools

import jax
import jax.numpy as jnp
from jax import lax
from jax.experimental import pallas as pl
from jax.experimental.pallas import tpu as pltpu
from jax.experimental.pallas import tpu_sc as plsc

_N = 10000
_E = 320000
_IN = 128
_OUT = 128
_R = 16
_B = 4

_TN = 400
_NT = _N // _TN

_NC = 2
_NS = 16
_NW = _NC * _NS
_EPW = 10000
_W = 80
_HW = _W // 2
_KW = _EPW // _W
_SW = 25
_NSC = _KW // _SW
_ND = 3
_NPAD = 10240
_RPS = _NPAD // _NS


def _h_body(x_ref, bases_ref, coeff_ref, wselft_ref, b_ref, h_ref, self_ref,
            wbig_ref):
    n = pl.program_id(0)

    @pl.when(n == 0)
    def _():
        for r in range(_R):
            w = (coeff_ref[r, 0] * bases_ref[0]
                 + coeff_ref[r, 1] * bases_ref[1]
                 + coeff_ref[r, 2] * bases_ref[2]
                 + coeff_ref[r, 3] * bases_ref[3])
            wbig_ref[:, r * _OUT:(r + 1) * _OUT] = w.astype(jnp.bfloat16)
        wbig_ref[:, _R * _OUT:] = wselft_ref[...].astype(jnp.bfloat16)

    x = x_ref[...].astype(jnp.bfloat16)
    h = lax.dot_general(x, wbig_ref[...], (((1,), (0,)), ((), ())),
                        preferred_element_type=jnp.float32)
    for r in range(_R):
        h_ref[r] = h[:, r * _OUT:(r + 1) * _OUT]
    self_ref[...] = h[:, _R * _OUT:] + b_ref[...]


def _h_pallas(x, bases, coeff, wself_t, b2d):
    return pl.pallas_call(
        _h_body,
        grid=(_NT,),
        in_specs=[
            pl.BlockSpec((_TN, _IN), lambda n: (n, 0)),
            pl.BlockSpec((_B, _IN, _OUT), lambda n: (0, 0, 0)),
            pl.BlockSpec(memory_space=pltpu.SMEM),
            pl.BlockSpec((_IN, _OUT), lambda n: (0, 0)),
            pl.BlockSpec((1, _OUT), lambda n: (0, 0)),
        ],
        out_specs=[
            pl.BlockSpec((_R, _TN, _OUT), lambda n: (0, n, 0)),
            pl.BlockSpec((_TN, _OUT), lambda n: (n, 0)),
        ],
        out_shape=[
            jax.ShapeDtypeStruct((_R, _N, _OUT), jnp.float32),
            jax.ShapeDtypeStruct((_N, _OUT), jnp.float32),
        ],
        scratch_shapes=[
            pltpu.VMEM((_IN, (_R + 1) * _OUT), jnp.bfloat16),
        ],
    )(x, bases, coeff, wself_t, b2d)


def _sc_body(h_hbm, ei_hbm, et_hbm, out_hbm,
             idx_v, et_v, tgt_v, gb0, gb1, gb2, acc_sh,
             sm00, sm01, sm10, sm11, sm20, sm21):
    c = lax.axis_index("c")
    s = lax.axis_index("s")
    wid = s * _NC + c
    bufs = [gb0, gb1, gb2]
    sems = [(sm00, sm01), (sm10, sm11), (sm20, sm21)]

    def _start_gather(j, b):
        pltpu.async_copy(h_hbm.at[idx_v.at[j, pl.ds(0, _HW)]],
                         bufs[b].at[pl.ds(0, _HW)], sems[b][0])
        pltpu.async_copy(h_hbm.at[idx_v.at[j, pl.ds(_HW, _HW)]],
                         bufs[b].at[pl.ds(_HW, _HW)], sems[b][1])

    def _finish_window(j, b):
        pltpu.make_async_copy(h_hbm.at[idx_v.at[j, pl.ds(0, _HW)]],
                              bufs[b].at[pl.ds(0, _HW)], sems[b][0]).wait()
        pltpu.make_async_copy(h_hbm.at[idx_v.at[j, pl.ds(_HW, _HW)]],
                              bufs[b].at[pl.ds(_HW, _HW)], sems[b][1]).wait()
        pltpu.sync_copy(bufs[b], acc_sh.at[tgt_v.at[j]], add=True)

    @pl.loop(0, _W)
    def _(rr):
        for cc in range(_OUT // 16):
            gb0[rr, pl.ds(cc * 16, 16)] = jnp.zeros((16,), jnp.float32)

    @pl.loop(0, _RPS // _W)
    def _(k):
        pltpu.sync_copy(gb0, acc_sh.at[pl.ds(s * _RPS + k * _W, _W)])
    plsc.subcore_barrier()

    @pl.loop(0, _NSC)
    def _(q):
        pltpu.sync_copy(ei_hbm.at[0, wid, q], idx_v)
        pltpu.sync_copy(et_hbm.at[wid, q], et_v)
        pltpu.sync_copy(ei_hbm.at[1, wid, q], tgt_v)

        @pl.loop(0, _SW)
        def _(j):
            for cc in range(_W // 16):
                sl = pl.ds(cc * 16, 16)
                idx_v[j, sl] = idx_v[j, sl] + et_v[j, sl] * _N

        for b in range(_ND):
            _start_gather(b, b)

        @pl.loop(0, _SW // _ND)
        def _(g):
            for b in range(_ND):
                j = _ND * g + b
                _finish_window(j, b)

                @pl.when(j + _ND < _SW)
                def _(b=b, j=j):
                    _start_gather(j + _ND, b)

        for j in range(_ND * (_SW // _ND), _SW):
            _finish_window(j, j % _ND)

    plsc.subcore_barrier()

    pltpu.sync_copy(acc_sh.at[pl.ds(s * _RPS, _RPS)],
                    out_hbm.at[c].at[pl.ds(s * _RPS, _RPS)])


_sc_scatter = functools.partial(
    pl.kernel,
    out_type=jax.ShapeDtypeStruct((_NC, _NPAD, _OUT), jnp.float32),
    mesh=plsc.VectorSubcoreMesh(core_axis_name="c", subcore_axis_name="s"),
    scratch_types=[
        pltpu.VMEM((_SW, _W), jnp.int32),
        pltpu.VMEM((_SW, _W), jnp.int32),
        pltpu.VMEM((_SW, _W), jnp.int32),
        pltpu.VMEM((_W, _OUT), jnp.float32),
        pltpu.VMEM((_W, _OUT), jnp.float32),
        pltpu.VMEM((_W, _OUT), jnp.float32),
        pltpu.VMEM_SHARED((_NPAD, _OUT), jnp.float32),
        pltpu.SemaphoreType.DMA,
        pltpu.SemaphoreType.DMA,
        pltpu.SemaphoreType.DMA,
        pltpu.SemaphoreType.DMA,
        pltpu.SemaphoreType.DMA,
        pltpu.SemaphoreType.DMA,
    ],
)(_sc_body)


def _relu_body(self_ref, acc_ref, out_ref):
    out_ref[...] = jnp.maximum(self_ref[...] + acc_ref[0] + acc_ref[1], 0.0)


_TR = 2000


def _relu_pallas(self_out, acc):
    return pl.pallas_call(
        _relu_body,
        grid=(_N // _TR,),
        in_specs=[
            pl.BlockSpec((_TR, _OUT), lambda n: (n, 0)),
            pl.BlockSpec((_NC, _TR, _OUT), lambda n: (0, n, 0)),
        ],
        out_specs=pl.BlockSpec((_TR, _OUT), lambda n: (n, 0)),
        out_shape=jax.ShapeDtypeStruct((_N, _OUT), jnp.float32),
    )(self_out, acc)


def kernel(node_features, edge_index, edge_type, W_self_w, W_self_b,
           bases, coefficients):
    h, self_out = _h_pallas(node_features, bases, coefficients,
                            W_self_w.T, W_self_b.reshape(1, _OUT))
    ei = edge_index.reshape(2, _NW, _NSC, _SW, _W)
    et = edge_type.reshape(_NW, _NSC, _SW, _W)
    acc = _sc_scatter(h.reshape(_R * _N, _OUT), ei, et)
    return _relu_pallas(self_out, acc)
